# baseline (speedup 1.0000x reference)
import functools

import jax
import jax.numpy as jnp
from jax import lax
from jax.experimental import pallas as pl
from jax.experimental.pallas import tpu as pltpu
from jax.experimental.pallas import tpu_sc as plsc

D = 64
NC, NS = 2, 16
NW = NC * NS
CR = 1


@jax.jit
def _embed(table, tokens):
    b, s = tokens.shape
    rows_per_w = b // NW
    n_chunks = rows_per_w // CR
    mesh = plsc.VectorSubcoreMesh(core_axis_name="c", subcore_axis_name="s")

    @functools.partial(
        pl.kernel,
        mesh=mesh,
        compiler_params=pltpu.CompilerParams(use_tc_tiling_on_sc=False),
        out_type=jax.ShapeDtypeStruct((b, s, D), jnp.float32),
        scratch_types=[
            pltpu.VMEM((rows_per_w, s), jnp.int32),
            pltpu.VMEM((2, s, D), jnp.float32),
            pltpu.SemaphoreType.DMA,
            pltpu.SemaphoreType.DMA,
            pltpu.SemaphoreType.DMA,
            pltpu.SemaphoreType.DMA,
        ],
    )
    def k(table_hbm, idx_hbm, out_hbm, idx_v, rows_v, g0, g1, s0, s1):
        wid = lax.axis_index("s") * NC + lax.axis_index("c")
        row0 = wid * rows_per_w
        gsem = (g0, g1)
        ssem = (s0, s1)

        pltpu.sync_copy(idx_hbm.at[pl.ds(row0, rows_per_w)], idx_v)

        def start_gather(j, bf):
            pltpu.async_copy(table_hbm.at[idx_v.at[j]], rows_v.at[bf], gsem[bf])

        def wait_gather(j, bf):
            pltpu.make_async_copy(
                table_hbm.at[idx_v.at[j]], rows_v.at[bf], gsem[bf]
            ).wait()

        def start_store(j, bf):
            pltpu.async_copy(rows_v.at[bf], out_hbm.at[row0 + j], ssem[bf])

        def wait_store(j, bf):
            pltpu.make_async_copy(
                rows_v.at[bf], out_hbm.at[row0 + j], ssem[bf]
            ).wait()

        start_gather(0, 0)

        @pl.loop(0, n_chunks, step=2)
        def _group(i):
            for bf in range(2):
                j = i + bf
                wait_gather(j, bf)

                @pl.when(j + 1 < n_chunks)
                def _():
                    @pl.when(j >= 1)
                    def _():
                        wait_store(j - 1, 1 - bf)

                    start_gather(j + 1, 1 - bf)

                start_store(j, bf)

        wait_store(n_chunks - 2, 0)
        wait_store(n_chunks - 1, 1)

    return k(table, tokens)


def kernel(tokens, table):
    return _embed(table, tokens.astype(jnp.int32))
